# Initial kernel scaffold; baseline (speedup 1.0000x reference)
import functools

import jax
import jax.numpy as jnp
from jax import lax
from jax.experimental import pallas as pl
from jax.experimental.pallas import tpu as pltpu
from jax.experimental.pallas import tpu_sc as plsc

N = 10000
E = 320000
F = 128
H = 64
G = 128

NCORE = 2
NSUB = 16
NW = NCORE * NSUB
EPT = E // NW
CH = 80
NCHUNK = EPT // CH
ROWS_PT = N // NSUB

_MESH = dict(core_axis_name="c", subcore_axis_name="s")


@functools.partial(
    pl.kernel,
    mesh=plsc.VectorSubcoreMesh(**_MESH),
    out_type=jax.ShapeDtypeStruct((NCORE, N, 16), jnp.float32),
    scratch_types=[
        pltpu.VMEM((NCHUNK, CH), jnp.int32),
        pltpu.VMEM((CH, 16), jnp.float32),
        pltpu.VMEM_SHARED((N, 16), jnp.float32),
    ],
)
def _sc_degree(dst_hbm, ones_hbm, zeros_hbm, out_hbm, dst_v, ones_v, acc_sh):
    c = lax.axis_index("c")
    s = lax.axis_index("s")
    w = c * NSUB + s
    pltpu.sync_copy(dst_hbm.at[w], dst_v)
    pltpu.sync_copy(ones_hbm, ones_v)
    pltpu.sync_copy(zeros_hbm.at[pl.ds(s * ROWS_PT, ROWS_PT)],
                    acc_sh.at[pl.ds(s * ROWS_PT, ROWS_PT)])
    plsc.subcore_barrier()

    def body(j, carry):
        pltpu.sync_copy(ones_v, acc_sh.at[dst_v.at[j]], add=True)
        return carry

    lax.fori_loop(0, NCHUNK, body, 0)
    plsc.subcore_barrier()
    pltpu.sync_copy(acc_sh.at[pl.ds(s * ROWS_PT, ROWS_PT)],
                    out_hbm.at[c, pl.ds(s * ROWS_PT, ROWS_PT)])


@functools.partial(
    pl.kernel,
    mesh=plsc.VectorSubcoreMesh(**_MESH),
    out_type=jax.ShapeDtypeStruct((NCORE, N, H), jnp.float32),
    scratch_types=[
        pltpu.VMEM((NCHUNK, CH), jnp.int32),
        pltpu.VMEM((NCHUNK, CH), jnp.int32),
        pltpu.VMEM((CH, H), jnp.float32),
        pltpu.VMEM_SHARED((N, H), jnp.float32),
        pltpu.SemaphoreType.DMA,
    ],
)
def _sc_agg(table_hbm, src_hbm, dst_hbm, zeros_hbm, out_hbm,
            src_v, dst_v, rows_v, acc_sh, sem):
    c = lax.axis_index("c")
    s = lax.axis_index("s")
    w = c * NSUB + s
    pltpu.sync_copy(src_hbm.at[w], src_v)
    pltpu.sync_copy(dst_hbm.at[w], dst_v)
    pltpu.sync_copy(zeros_hbm.at[pl.ds(s * ROWS_PT, ROWS_PT)],
                    acc_sh.at[pl.ds(s * ROWS_PT, ROWS_PT)])
    plsc.subcore_barrier()

    def body(j, carry):
        pltpu.async_copy(table_hbm.at[src_v.at[j]], rows_v, sem).wait()
        pltpu.sync_copy(rows_v, acc_sh.at[dst_v.at[j]], add=True)
        return carry

    lax.fori_loop(0, NCHUNK, body, 0)
    plsc.subcore_barrier()
    pltpu.sync_copy(acc_sh.at[pl.ds(s * ROWS_PT, ROWS_PT)],
                    out_hbm.at[c, pl.ds(s * ROWS_PT, ROWS_PT)])


def _din(dc0_ref, dc1_ref):
    deg = dc0_ref[...] + dc1_ref[...] + 1.0
    return lax.rsqrt(deg[:, :1])


def _tc_first_body(x_ref, w_ref, dc0_ref, dc1_ref, hwp_ref):
    din = _din(dc0_ref, dc1_ref)
    hw = jnp.dot(x_ref[...], w_ref[...], preferred_element_type=jnp.float32)
    hwp_ref[...] = hw * din


_tc_first = pl.pallas_call(
    _tc_first_body,
    out_shape=jax.ShapeDtypeStruct((N, H), jnp.float32),
)


def _tc_mid_body(acc0_ref, acc1_ref, hwp_ref, dc0_ref, dc1_ref, b_ref, w_ref,
                 out_ref):
    din = _din(dc0_ref, dc1_ref)
    h = jnp.tanh(din * (acc0_ref[...] + acc1_ref[...] + hwp_ref[...])
                 + b_ref[...])
    out_ref[...] = jnp.dot(h, w_ref[...],
                           preferred_element_type=jnp.float32) * din


_tc_mid = pl.pallas_call(
    _tc_mid_body,
    out_shape=jax.ShapeDtypeStruct((N, H), jnp.float32),
)


def _tc_pool_body(acc0_ref, acc1_ref, hwp_ref, dc0_ref, dc1_ref, b_ref,
                  bi_ref, wout_ref, bout_ref, out_ref, hidden_ref):
    din = _din(dc0_ref, dc1_ref)
    h = jnp.tanh(din * (acc0_ref[...] + acc1_ref[...] + hwp_ref[...])
                 + b_ref[...])
    bi = bi_ref[...]
    onehot = (bi == lax.broadcasted_iota(jnp.int32, (1, G), 1)
              ).astype(jnp.float32)
    sums = lax.dot_general(onehot, h, (((0,), (0,)), ((), ())),
                           preferred_element_type=jnp.float32)
    counts = lax.dot_general(onehot, jnp.ones((N, 1), jnp.float32),
                             (((0,), (0,)), ((), ())),
                             preferred_element_type=jnp.float32)
    neg = jnp.float32(-jnp.inf)

    def mbody(g, gmp):
        mx = jnp.max(jnp.where(bi == g, h, neg), axis=0, keepdims=True)
        return lax.dynamic_update_slice(gmp, mx, (g, 0))

    gmp = lax.fori_loop(0, G, mbody, jnp.full((G, H), neg, jnp.float32))
    gmp = jnp.where(counts > 0.0, gmp, 0.0)
    gap = sums / jnp.maximum(counts, 1.0)
    hidden = jnp.concatenate([gmp, gap], axis=1)
    out_ref[...] = jnp.dot(hidden, wout_ref[...],
                           preferred_element_type=jnp.float32) + bout_ref[...]
    hidden_ref[...] = hidden


_tc_pool = pl.pallas_call(
    _tc_pool_body,
    out_shape=(jax.ShapeDtypeStruct((G, 1), jnp.float32),
               jax.ShapeDtypeStruct((G, 2 * H), jnp.float32)),
)


def kernel(x, edge_index, batch_index, W0, b0, W1, b1, W2, b2, W3, b3,
           Wout, bout):
    src = edge_index[0].reshape(NW, NCHUNK, CH)
    dst = edge_index[1].reshape(NW, NCHUNK, CH)
    ones16 = jnp.ones((CH, 16), jnp.float32)
    zeros16 = jnp.zeros((N, 16), jnp.float32)
    zerosH = jnp.zeros((N, H), jnp.float32)

    degcnt = _sc_degree(dst, ones16, zeros16)
    dc0, dc1 = degcnt[0], degcnt[1]

    hwp = _tc_first(x, W0, dc0, dc1)
    for (b, W) in ((b0, W1), (b1, W2), (b2, W3)):
        acc = _sc_agg(hwp, src, dst, zerosH)
        hwp = _tc_mid(acc[0], acc[1], hwp, dc0, dc1, b.reshape(1, H), W)
    acc = _sc_agg(hwp, src, dst, zerosH)
    out, hidden = _tc_pool(acc[0], acc[1], hwp, dc0, dc1, b3.reshape(1, H),
                           batch_index.reshape(N, 1), Wout,
                           bout.reshape(1, 1))
    return (out, hidden)


# speedup vs baseline: 14.8171x; 14.8171x over previous
import functools

import jax
import jax.numpy as jnp
from jax import lax
from jax.experimental import pallas as pl
from jax.experimental.pallas import tpu as pltpu
from jax.experimental.pallas import tpu_sc as plsc

N = 10000
E = 320000
F = 128
H = 64
G = 128

NCORE = 2
NSUB = 16
NW = NCORE * NSUB
EPT = E // NW
CH = 80
NCHUNK = EPT // CH
NP = 10112
ROWS_PT = NP // NSUB

_MESH = dict(core_axis_name="c", subcore_axis_name="s")


@functools.partial(
    pl.kernel,
    mesh=plsc.VectorSubcoreMesh(**_MESH),
    compiler_params=pltpu.CompilerParams(use_tc_tiling_on_sc=False),
    out_type=jax.ShapeDtypeStruct((NCORE, NP, 16), jnp.float32),
    scratch_types=[
        pltpu.VMEM((NCHUNK, CH), jnp.int32),
        pltpu.VMEM((CH, 16), jnp.float32),
        pltpu.VMEM_SHARED((NP, 16), jnp.float32),
    ],
)
def _sc_degree(dst_hbm, ones_hbm, zeros_hbm, out_hbm, dst_v, ones_v, acc_sh):
    c = lax.axis_index("c")
    s = lax.axis_index("s")
    w = c * NSUB + s
    pltpu.sync_copy(dst_hbm.at[w], dst_v)
    pltpu.sync_copy(ones_hbm, ones_v)
    pltpu.sync_copy(zeros_hbm.at[pl.ds(s * ROWS_PT, ROWS_PT)],
                    acc_sh.at[pl.ds(s * ROWS_PT, ROWS_PT)])
    plsc.subcore_barrier()

    def body(j, carry):
        pltpu.sync_copy(ones_v, acc_sh.at[dst_v.at[j]], add=True)
        return carry

    lax.fori_loop(0, NCHUNK, body, 0)
    plsc.subcore_barrier()
    pltpu.sync_copy(acc_sh.at[pl.ds(s * ROWS_PT, ROWS_PT)],
                    out_hbm.at[c, pl.ds(s * ROWS_PT, ROWS_PT)])


@functools.partial(
    pl.kernel,
    mesh=plsc.VectorSubcoreMesh(**_MESH),
    compiler_params=pltpu.CompilerParams(use_tc_tiling_on_sc=False),
    out_type=jax.ShapeDtypeStruct((NCORE, NP, H), jnp.float32),
    scratch_types=[
        pltpu.VMEM((NCHUNK, CH), jnp.int32),
        pltpu.VMEM((NCHUNK, CH), jnp.int32),
        pltpu.VMEM((CH, H), jnp.float32),
        pltpu.VMEM_SHARED((NP, H), jnp.float32),
        pltpu.SemaphoreType.DMA,
    ],
)
def _sc_agg(table_hbm, src_hbm, dst_hbm, zeros_hbm, out_hbm,
            src_v, dst_v, rows_v, acc_sh, sem):
    c = lax.axis_index("c")
    s = lax.axis_index("s")
    w = c * NSUB + s
    pltpu.sync_copy(src_hbm.at[w], src_v)
    pltpu.sync_copy(dst_hbm.at[w], dst_v)
    pltpu.sync_copy(zeros_hbm.at[pl.ds(s * ROWS_PT, ROWS_PT)],
                    acc_sh.at[pl.ds(s * ROWS_PT, ROWS_PT)])
    plsc.subcore_barrier()

    def body(j, carry):
        pltpu.async_copy(table_hbm.at[src_v.at[j]], rows_v, sem).wait()
        pltpu.sync_copy(rows_v, acc_sh.at[dst_v.at[j]], add=True)
        return carry

    lax.fori_loop(0, NCHUNK, body, 0)
    plsc.subcore_barrier()
    pltpu.sync_copy(acc_sh.at[pl.ds(s * ROWS_PT, ROWS_PT)],
                    out_hbm.at[c, pl.ds(s * ROWS_PT, ROWS_PT)])


def _din(dc0_ref, dc1_ref):
    deg = dc0_ref[...][:N] + dc1_ref[...][:N] + 1.0
    return lax.rsqrt(deg[:, :1])


def _tc_first_body(x_ref, w_ref, dc0_ref, dc1_ref, hwp_ref):
    din = _din(dc0_ref, dc1_ref)
    hw = jnp.dot(x_ref[...], w_ref[...], preferred_element_type=jnp.float32)
    hwp_ref[...] = hw * din


_tc_first = pl.pallas_call(
    _tc_first_body,
    out_shape=jax.ShapeDtypeStruct((N, H), jnp.float32),
)


def _tc_mid_body(acc0_ref, acc1_ref, hwp_ref, dc0_ref, dc1_ref, b_ref, w_ref,
                 out_ref):
    din = _din(dc0_ref, dc1_ref)
    h = jnp.tanh(din * (acc0_ref[...][:N] + acc1_ref[...][:N] + hwp_ref[...])
                 + b_ref[...])
    out_ref[...] = jnp.dot(h, w_ref[...],
                           preferred_element_type=jnp.float32) * din


_tc_mid = pl.pallas_call(
    _tc_mid_body,
    out_shape=jax.ShapeDtypeStruct((N, H), jnp.float32),
)


def _tc_pool_body(acc0_ref, acc1_ref, hwp_ref, dc0_ref, dc1_ref, b_ref,
                  bi_ref, wout_ref, bout_ref, out_ref, hidden_ref):
    din = _din(dc0_ref, dc1_ref)
    h = jnp.tanh(din * (acc0_ref[...][:N] + acc1_ref[...][:N] + hwp_ref[...])
                 + b_ref[...])
    bi = bi_ref[...]
    onehot = (bi == lax.broadcasted_iota(jnp.int32, (1, G), 1)
              ).astype(jnp.float32)
    sums = lax.dot_general(onehot, h, (((0,), (0,)), ((), ())),
                           preferred_element_type=jnp.float32)
    counts = lax.dot_general(onehot, jnp.ones((N, 1), jnp.float32),
                             (((0,), (0,)), ((), ())),
                             preferred_element_type=jnp.float32)
    neg = jnp.float32(-jnp.inf)

    def mbody(g, carry):
        mx = jnp.max(jnp.where(bi == g, h, neg), axis=0, keepdims=True)
        hidden_ref[pl.ds(g, 1), 0:H] = mx
        return carry

    lax.fori_loop(0, G, mbody, 0)
    gmp = jnp.where(counts > 0.0, hidden_ref[:, 0:H], 0.0)
    gap = sums / jnp.maximum(counts, 1.0)
    hidden = jnp.concatenate([gmp, gap], axis=1)
    out_ref[...] = jnp.dot(hidden, wout_ref[...],
                           preferred_element_type=jnp.float32) + bout_ref[...]
    hidden_ref[...] = hidden


_tc_pool = pl.pallas_call(
    _tc_pool_body,
    out_shape=(jax.ShapeDtypeStruct((G, 1), jnp.float32),
               jax.ShapeDtypeStruct((G, 2 * H), jnp.float32)),
)


def kernel(x, edge_index, batch_index, W0, b0, W1, b1, W2, b2, W3, b3,
           Wout, bout):
    src = edge_index[0].reshape(NW, NCHUNK, CH)
    dst = edge_index[1].reshape(NW, NCHUNK, CH)
    ones16 = jnp.ones((CH, 16), jnp.float32)
    zeros16 = jnp.zeros((NP, 16), jnp.float32)
    zerosH = jnp.zeros((NP, H), jnp.float32)

    degcnt = _sc_degree(dst, ones16, zeros16)
    dc0, dc1 = degcnt[0], degcnt[1]

    hwp = _tc_first(x, W0, dc0, dc1)
    for (b, W) in ((b0, W1), (b1, W2), (b2, W3)):
        acc = _sc_agg(hwp, src, dst, zerosH)
        hwp = _tc_mid(acc[0], acc[1], hwp, dc0, dc1, b.reshape(1, H), W)
    acc = _sc_agg(hwp, src, dst, zerosH)
    out, hidden = _tc_pool(acc[0], acc[1], hwp, dc0, dc1, b3.reshape(1, H),
                           batch_index.reshape(N, 1), Wout,
                           bout.reshape(1, 1))
    return (out, hidden)


# speedup vs baseline: 20.0849x; 1.3555x over previous
import functools

import jax
import jax.numpy as jnp
from jax import lax
from jax.experimental import pallas as pl
from jax.experimental.pallas import tpu as pltpu
from jax.experimental.pallas import tpu_sc as plsc

N = 10000
E = 320000
F = 128
H = 64
G = 128

NCORE = 2
NSUB = 16
NW = NCORE * NSUB
EPT = E // NW
CH = 80
NCHUNK = EPT // CH
NP = 10112
ROWS_PT = NP // NSUB

_MESH = dict(core_axis_name="c", subcore_axis_name="s")


@functools.partial(
    pl.kernel,
    mesh=plsc.VectorSubcoreMesh(**_MESH),
    compiler_params=pltpu.CompilerParams(use_tc_tiling_on_sc=False),
    out_type=jax.ShapeDtypeStruct((NCORE, NP, 16), jnp.float32),
    scratch_types=[
        pltpu.VMEM((NCHUNK, CH), jnp.int32),
        pltpu.VMEM((CH, 16), jnp.float32),
        pltpu.VMEM_SHARED((NP, 16), jnp.float32),
    ],
)
def _sc_degree(dst_hbm, ones_hbm, zeros_hbm, out_hbm, dst_v, ones_v, acc_sh):
    c = lax.axis_index("c")
    s = lax.axis_index("s")
    w = c * NSUB + s
    pltpu.sync_copy(dst_hbm.at[w], dst_v)
    pltpu.sync_copy(ones_hbm, ones_v)
    pltpu.sync_copy(zeros_hbm.at[pl.ds(s * ROWS_PT, ROWS_PT)],
                    acc_sh.at[pl.ds(s * ROWS_PT, ROWS_PT)])
    plsc.subcore_barrier()

    def body(j, carry):
        pltpu.sync_copy(ones_v, acc_sh.at[dst_v.at[j]], add=True)
        return carry

    lax.fori_loop(0, NCHUNK, body, 0)
    plsc.subcore_barrier()
    pltpu.sync_copy(acc_sh.at[pl.ds(s * ROWS_PT, ROWS_PT)],
                    out_hbm.at[c, pl.ds(s * ROWS_PT, ROWS_PT)])


@functools.partial(
    pl.kernel,
    mesh=plsc.VectorSubcoreMesh(**_MESH),
    compiler_params=pltpu.CompilerParams(use_tc_tiling_on_sc=False),
    out_type=jax.ShapeDtypeStruct((NCORE, NP, H), jnp.float32),
    scratch_types=[
        pltpu.VMEM((NCHUNK, CH), jnp.int32),
        pltpu.VMEM((NCHUNK, CH), jnp.int32),
        pltpu.VMEM((CH, H), jnp.float32),
        pltpu.VMEM((CH, H), jnp.float32),
        pltpu.VMEM_SHARED((NP, H), jnp.float32),
        pltpu.SemaphoreType.DMA,
        pltpu.SemaphoreType.DMA,
    ],
)
def _sc_agg(table_hbm, src_hbm, dst_hbm, zeros_hbm, out_hbm,
            src_v, dst_v, rows0_v, rows1_v, acc_sh, sem0, sem1):
    c = lax.axis_index("c")
    s = lax.axis_index("s")
    w = c * NSUB + s
    pltpu.sync_copy(src_hbm.at[w], src_v)
    pltpu.sync_copy(dst_hbm.at[w], dst_v)
    pltpu.sync_copy(zeros_hbm.at[pl.ds(s * ROWS_PT, ROWS_PT)],
                    acc_sh.at[pl.ds(s * ROWS_PT, ROWS_PT)])
    plsc.subcore_barrier()

    pltpu.async_copy(table_hbm.at[src_v.at[0]], rows0_v, sem0)
    pltpu.async_copy(table_hbm.at[src_v.at[1]], rows1_v, sem1)

    def body(i, carry):
        j0 = 2 * i
        pltpu.make_async_copy(table_hbm.at[src_v.at[j0]], rows0_v,
                              sem0).wait()
        pltpu.sync_copy(rows0_v, acc_sh.at[dst_v.at[j0]], add=True)
        pltpu.async_copy(table_hbm.at[src_v.at[j0 + 2]], rows0_v, sem0)
        j1 = j0 + 1
        pltpu.make_async_copy(table_hbm.at[src_v.at[j1]], rows1_v,
                              sem1).wait()
        pltpu.sync_copy(rows1_v, acc_sh.at[dst_v.at[j1]], add=True)

        @pl.when(i < (NCHUNK - 1) // 2 - 1)
        def _():
            pltpu.async_copy(table_hbm.at[src_v.at[j1 + 2]], rows1_v, sem1)

        return carry

    lax.fori_loop(0, (NCHUNK - 1) // 2, body, 0)
    pltpu.make_async_copy(table_hbm.at[src_v.at[NCHUNK - 1]], rows0_v,
                          sem0).wait()
    pltpu.sync_copy(rows0_v, acc_sh.at[dst_v.at[NCHUNK - 1]], add=True)
    plsc.subcore_barrier()
    pltpu.sync_copy(acc_sh.at[pl.ds(s * ROWS_PT, ROWS_PT)],
                    out_hbm.at[c, pl.ds(s * ROWS_PT, ROWS_PT)])


def _din(dc0_ref, dc1_ref):
    deg = dc0_ref[...][:N] + dc1_ref[...][:N] + 1.0
    return lax.rsqrt(deg[:, :1])


def _tc_first_body(x_ref, w_ref, dc0_ref, dc1_ref, hwp_ref):
    din = _din(dc0_ref, dc1_ref)
    hw = jnp.dot(x_ref[...], w_ref[...], preferred_element_type=jnp.float32)
    hwp_ref[...] = hw * din


_tc_first = pl.pallas_call(
    _tc_first_body,
    out_shape=jax.ShapeDtypeStruct((N, H), jnp.float32),
)


def _tc_mid_body(acc0_ref, acc1_ref, hwp_ref, dc0_ref, dc1_ref, b_ref, w_ref,
                 out_ref):
    din = _din(dc0_ref, dc1_ref)
    h = jnp.tanh(din * (acc0_ref[...][:N] + acc1_ref[...][:N] + hwp_ref[...])
                 + b_ref[...])
    out_ref[...] = jnp.dot(h, w_ref[...],
                           preferred_element_type=jnp.float32) * din


_tc_mid = pl.pallas_call(
    _tc_mid_body,
    out_shape=jax.ShapeDtypeStruct((N, H), jnp.float32),
)


def _tc_pool_body(acc0_ref, acc1_ref, hwp_ref, dc0_ref, dc1_ref, b_ref,
                  bi_ref, wout_ref, bout_ref, out_ref, hidden_ref):
    din = _din(dc0_ref, dc1_ref)
    h = jnp.tanh(din * (acc0_ref[...][:N] + acc1_ref[...][:N] + hwp_ref[...])
                 + b_ref[...])
    bi = bi_ref[...]
    onehot = (bi == lax.broadcasted_iota(jnp.int32, (1, G), 1)
              ).astype(jnp.float32)
    sums = lax.dot_general(onehot, h, (((0,), (0,)), ((), ())),
                           preferred_element_type=jnp.float32)
    counts = lax.dot_general(onehot, jnp.ones((N, 1), jnp.float32),
                             (((0,), (0,)), ((), ())),
                             preferred_element_type=jnp.float32)
    neg = jnp.float32(-jnp.inf)

    def mbody(g, carry):
        mx = jnp.max(jnp.where(bi == g, h, neg), axis=0, keepdims=True)
        hidden_ref[pl.ds(g, 1), 0:H] = mx
        return carry

    lax.fori_loop(0, G, mbody, 0)
    gmp = jnp.where(counts > 0.0, hidden_ref[:, 0:H], 0.0)
    gap = sums / jnp.maximum(counts, 1.0)
    hidden = jnp.concatenate([gmp, gap], axis=1)
    out_ref[...] = jnp.dot(hidden, wout_ref[...],
                           preferred_element_type=jnp.float32) + bout_ref[...]
    hidden_ref[...] = hidden


_tc_pool = pl.pallas_call(
    _tc_pool_body,
    out_shape=(jax.ShapeDtypeStruct((G, 1), jnp.float32),
               jax.ShapeDtypeStruct((G, 2 * H), jnp.float32)),
)


def kernel(x, edge_index, batch_index, W0, b0, W1, b1, W2, b2, W3, b3,
           Wout, bout):
    src = edge_index[0].reshape(NW, NCHUNK, CH)
    dst = edge_index[1].reshape(NW, NCHUNK, CH)
    ones16 = jnp.ones((CH, 16), jnp.float32)
    zeros16 = jnp.zeros((NP, 16), jnp.float32)
    zerosH = jnp.zeros((NP, H), jnp.float32)

    degcnt = _sc_degree(dst, ones16, zeros16)
    dc0, dc1 = degcnt[0], degcnt[1]

    hwp = _tc_first(x, W0, dc0, dc1)
    for (b, W) in ((b0, W1), (b1, W2), (b2, W3)):
        acc = _sc_agg(hwp, src, dst, zerosH)
        hwp = _tc_mid(acc[0], acc[1], hwp, dc0, dc1, b.reshape(1, H), W)
    acc = _sc_agg(hwp, src, dst, zerosH)
    out, hidden = _tc_pool(acc[0], acc[1], hwp, dc0, dc1, b3.reshape(1, H),
                           batch_index.reshape(N, 1), Wout,
                           bout.reshape(1, 1))
    return (out, hidden)


# speedup vs baseline: 26.2048x; 1.3047x over previous
import functools

import jax
import jax.numpy as jnp
from jax import lax
from jax.experimental import pallas as pl
from jax.experimental.pallas import tpu as pltpu
from jax.experimental.pallas import tpu_sc as plsc

N = 10000
E = 320000
F = 128
H = 64
G = 128

NCORE = 2
NSUB = 16
NW = NCORE * NSUB
EPT = E // NW
CH = 80
NCHUNK = EPT // CH
NP = 10112
ROWS_PT = NP // NSUB

_MESH = dict(core_axis_name="c", subcore_axis_name="s")


@functools.partial(
    pl.kernel,
    mesh=plsc.VectorSubcoreMesh(**_MESH),
    compiler_params=pltpu.CompilerParams(use_tc_tiling_on_sc=False),
    out_type=jax.ShapeDtypeStruct((NCORE, NP, 16), jnp.float32),
    scratch_types=[
        pltpu.VMEM((NCHUNK, CH), jnp.int32),
        pltpu.VMEM((CH, 16), jnp.float32),
        pltpu.VMEM_SHARED((NP, 16), jnp.float32),
    ],
)
def _sc_degree(dst_hbm, ones_hbm, zeros_hbm, out_hbm, dst_v, ones_v, acc_sh):
    c = lax.axis_index("c")
    s = lax.axis_index("s")
    w = c * NSUB + s
    pltpu.sync_copy(dst_hbm.at[w], dst_v)
    pltpu.sync_copy(ones_hbm, ones_v)
    pltpu.sync_copy(zeros_hbm.at[pl.ds(s * ROWS_PT, ROWS_PT)],
                    acc_sh.at[pl.ds(s * ROWS_PT, ROWS_PT)])
    plsc.subcore_barrier()

    def body(j, carry):
        pltpu.sync_copy(ones_v, acc_sh.at[dst_v.at[j]], add=True)
        return carry

    lax.fori_loop(0, NCHUNK, body, 0)
    plsc.subcore_barrier()
    pltpu.sync_copy(acc_sh.at[pl.ds(s * ROWS_PT, ROWS_PT)],
                    out_hbm.at[c, pl.ds(s * ROWS_PT, ROWS_PT)])


@functools.partial(
    pl.kernel,
    mesh=plsc.VectorSubcoreMesh(**_MESH),
    compiler_params=pltpu.CompilerParams(use_tc_tiling_on_sc=False),
    out_type=jax.ShapeDtypeStruct((NCORE, NP, H), jnp.float32),
    scratch_types=[
        pltpu.VMEM((NCHUNK, CH), jnp.int32),
        pltpu.VMEM((NCHUNK, CH), jnp.int32),
        pltpu.VMEM((CH, H), jnp.float32),
        pltpu.VMEM((CH, H), jnp.float32),
        pltpu.VMEM_SHARED((NP, H), jnp.float32),
        pltpu.VMEM_SHARED((N, H), jnp.float32),
        pltpu.SemaphoreType.DMA,
        pltpu.SemaphoreType.DMA,
    ],
)
def _sc_agg(table_hbm, src_hbm, dst_hbm, zeros_hbm, out_hbm,
            src_v, dst_v, rows0_v, rows1_v, acc_sh, table_sh, sem0, sem1):
    c = lax.axis_index("c")
    s = lax.axis_index("s")
    w = c * NSUB + s
    pltpu.sync_copy(src_hbm.at[w], src_v)
    pltpu.sync_copy(dst_hbm.at[w], dst_v)
    pltpu.sync_copy(zeros_hbm.at[pl.ds(s * ROWS_PT, ROWS_PT)],
                    acc_sh.at[pl.ds(s * ROWS_PT, ROWS_PT)])
    pltpu.sync_copy(table_hbm.at[pl.ds(s * (N // NSUB), N // NSUB)],
                    table_sh.at[pl.ds(s * (N // NSUB), N // NSUB)])
    plsc.subcore_barrier()

    pltpu.async_copy(table_sh.at[src_v.at[0]], rows0_v, sem0)
    pltpu.async_copy(table_sh.at[src_v.at[1]], rows1_v, sem1)

    def body(i, carry):
        j0 = 2 * i
        pltpu.make_async_copy(table_sh.at[src_v.at[j0]], rows0_v,
                              sem0).wait()
        pltpu.sync_copy(rows0_v, acc_sh.at[dst_v.at[j0]], add=True)
        pltpu.async_copy(table_sh.at[src_v.at[j0 + 2]], rows0_v, sem0)
        j1 = j0 + 1
        pltpu.make_async_copy(table_sh.at[src_v.at[j1]], rows1_v,
                              sem1).wait()
        pltpu.sync_copy(rows1_v, acc_sh.at[dst_v.at[j1]], add=True)

        @pl.when(i < (NCHUNK - 1) // 2 - 1)
        def _():
            pltpu.async_copy(table_sh.at[src_v.at[j1 + 2]], rows1_v, sem1)

        return carry

    lax.fori_loop(0, (NCHUNK - 1) // 2, body, 0)
    pltpu.make_async_copy(table_sh.at[src_v.at[NCHUNK - 1]], rows0_v,
                          sem0).wait()
    pltpu.sync_copy(rows0_v, acc_sh.at[dst_v.at[NCHUNK - 1]], add=True)
    plsc.subcore_barrier()
    pltpu.sync_copy(acc_sh.at[pl.ds(s * ROWS_PT, ROWS_PT)],
                    out_hbm.at[c, pl.ds(s * ROWS_PT, ROWS_PT)])


def _din(dc0_ref, dc1_ref):
    deg = dc0_ref[...][:N] + dc1_ref[...][:N] + 1.0
    return lax.rsqrt(deg[:, :1])


def _tc_first_body(x_ref, w_ref, dc0_ref, dc1_ref, hwp_ref):
    din = _din(dc0_ref, dc1_ref)
    hw = jnp.dot(x_ref[...], w_ref[...], preferred_element_type=jnp.float32)
    hwp_ref[...] = hw * din


_tc_first = pl.pallas_call(
    _tc_first_body,
    out_shape=jax.ShapeDtypeStruct((N, H), jnp.float32),
)


def _tc_mid_body(acc0_ref, acc1_ref, hwp_ref, dc0_ref, dc1_ref, b_ref, w_ref,
                 out_ref):
    din = _din(dc0_ref, dc1_ref)
    h = jnp.tanh(din * (acc0_ref[...][:N] + acc1_ref[...][:N] + hwp_ref[...])
                 + b_ref[...])
    out_ref[...] = jnp.dot(h, w_ref[...],
                           preferred_element_type=jnp.float32) * din


_tc_mid = pl.pallas_call(
    _tc_mid_body,
    out_shape=jax.ShapeDtypeStruct((N, H), jnp.float32),
)


def _tc_pool_body(acc0_ref, acc1_ref, hwp_ref, dc0_ref, dc1_ref, b_ref,
                  bi_ref, wout_ref, bout_ref, out_ref, hidden_ref):
    din = _din(dc0_ref, dc1_ref)
    h = jnp.tanh(din * (acc0_ref[...][:N] + acc1_ref[...][:N] + hwp_ref[...])
                 + b_ref[...])
    bi = bi_ref[...]
    onehot = (bi == lax.broadcasted_iota(jnp.int32, (1, G), 1)
              ).astype(jnp.float32)
    sums = lax.dot_general(onehot, h, (((0,), (0,)), ((), ())),
                           preferred_element_type=jnp.float32)
    counts = lax.dot_general(onehot, jnp.ones((N, 1), jnp.float32),
                             (((0,), (0,)), ((), ())),
                             preferred_element_type=jnp.float32)
    neg = jnp.float32(-jnp.inf)

    m = h
    d = 1
    while d < N:
        ids_sh = jnp.concatenate(
            [jnp.full((d, 1), -1, jnp.int32), bi[:N - d]], axis=0)
        m_sh = jnp.concatenate(
            [jnp.full((d, H), neg, jnp.float32), m[:N - d]], axis=0)
        m = jnp.maximum(m, jnp.where(ids_sh == bi, m_sh, neg))
        d *= 2
    nxt = jnp.concatenate([bi[1:], jnp.full((1, 1), -1, jnp.int32)], axis=0)
    wlast = onehot * (bi != nxt).astype(jnp.float32)
    gmp = lax.dot_general(wlast, m, (((0,), (0,)), ((), ())),
                          preferred_element_type=jnp.float32)
    gmp = jnp.where(counts > 0.0, gmp, 0.0)
    gap = sums / jnp.maximum(counts, 1.0)
    hidden = jnp.concatenate([gmp, gap], axis=1)
    out_ref[...] = jnp.dot(hidden, wout_ref[...],
                           preferred_element_type=jnp.float32) + bout_ref[...]
    hidden_ref[...] = hidden


_tc_pool = pl.pallas_call(
    _tc_pool_body,
    out_shape=(jax.ShapeDtypeStruct((G, 1), jnp.float32),
               jax.ShapeDtypeStruct((G, 2 * H), jnp.float32)),
    compiler_params=pltpu.CompilerParams(vmem_limit_bytes=100 * 1024 * 1024),
)


def kernel(x, edge_index, batch_index, W0, b0, W1, b1, W2, b2, W3, b3,
           Wout, bout):
    src = edge_index[0].reshape(NW, NCHUNK, CH)
    dst = edge_index[1].reshape(NW, NCHUNK, CH)
    ones16 = jnp.ones((CH, 16), jnp.float32)
    zeros16 = jnp.zeros((NP, 16), jnp.float32)
    zerosH = jnp.zeros((NP, H), jnp.float32)

    degcnt = _sc_degree(dst, ones16, zeros16)
    dc0, dc1 = degcnt[0], degcnt[1]

    hwp = _tc_first(x, W0, dc0, dc1)
    for (b, W) in ((b0, W1), (b1, W2), (b2, W3)):
        acc = _sc_agg(hwp, src, dst, zerosH)
        hwp = _tc_mid(acc[0], acc[1], hwp, dc0, dc1, b.reshape(1, H), W)
    acc = _sc_agg(hwp, src, dst, zerosH)
    out, hidden = _tc_pool(acc[0], acc[1], hwp, dc0, dc1, b3.reshape(1, H),
                           batch_index.reshape(N, 1), Wout,
                           bout.reshape(1, 1))
    return (out, hidden)


# speedup vs baseline: 29.2301x; 1.1154x over previous
import functools

import jax
import jax.numpy as jnp
from jax import lax
from jax.experimental import pallas as pl
from jax.experimental.pallas import tpu as pltpu
from jax.experimental.pallas import tpu_sc as plsc

N = 10000
E = 320000
F = 128
H = 64
G = 128

NCORE = 2
NSUB = 16
NW = NCORE * NSUB
EPT = E // NW
CH = 80
NCHUNK = EPT // CH
NP = 10112
ROWS_PT = NP // NSUB

_MESH = dict(core_axis_name="c", subcore_axis_name="s")


@functools.partial(
    pl.kernel,
    mesh=plsc.VectorSubcoreMesh(**_MESH),
    compiler_params=pltpu.CompilerParams(use_tc_tiling_on_sc=False),
    out_type=jax.ShapeDtypeStruct((NCORE, NP, 16), jnp.float32),
    scratch_types=[
        pltpu.VMEM((NCHUNK, CH), jnp.int32),
        pltpu.VMEM((CH, 16), jnp.float32),
        pltpu.VMEM_SHARED((NP, 16), jnp.float32),
    ],
)
def _sc_degree(dst_hbm, ones_hbm, zeros_hbm, out_hbm, dst_v, ones_v, acc_sh):
    c = lax.axis_index("c")
    s = lax.axis_index("s")
    w = c * NSUB + s
    pltpu.sync_copy(dst_hbm.at[w], dst_v)
    pltpu.sync_copy(ones_hbm, ones_v)
    pltpu.sync_copy(zeros_hbm.at[pl.ds(s * ROWS_PT, ROWS_PT)],
                    acc_sh.at[pl.ds(s * ROWS_PT, ROWS_PT)])
    plsc.subcore_barrier()

    def body(j, carry):
        pltpu.sync_copy(ones_v, acc_sh.at[dst_v.at[j]], add=True)
        return carry

    lax.fori_loop(0, NCHUNK, body, 0)
    plsc.subcore_barrier()
    pltpu.sync_copy(acc_sh.at[pl.ds(s * ROWS_PT, ROWS_PT)],
                    out_hbm.at[c, pl.ds(s * ROWS_PT, ROWS_PT)])


@functools.partial(
    pl.kernel,
    mesh=plsc.VectorSubcoreMesh(**_MESH),
    compiler_params=pltpu.CompilerParams(use_tc_tiling_on_sc=False),
    out_type=jax.ShapeDtypeStruct((NCORE, NP, H), jnp.float32),
    scratch_types=[
        pltpu.VMEM((NCHUNK, CH), jnp.int32),
        pltpu.VMEM((NCHUNK, CH), jnp.int32),
        pltpu.VMEM((CH, H), jnp.float32),
        pltpu.VMEM((CH, H), jnp.float32),
        pltpu.VMEM((CH, H), jnp.float32),
        pltpu.VMEM((CH, H), jnp.float32),
        pltpu.VMEM_SHARED((NP, H), jnp.float32),
        pltpu.VMEM_SHARED((N, H), jnp.float32),
        pltpu.SemaphoreType.DMA,
        pltpu.SemaphoreType.DMA,
        pltpu.SemaphoreType.DMA,
        pltpu.SemaphoreType.DMA,
        pltpu.SemaphoreType.DMA,
        pltpu.SemaphoreType.DMA,
        pltpu.SemaphoreType.DMA,
        pltpu.SemaphoreType.DMA,
    ],
)
def _sc_agg(table_hbm, src_hbm, dst_hbm, zeros_hbm, out_hbm,
            src_v, dst_v, rows0_v, rows1_v, rows2_v, rows3_v,
            acc_sh, table_sh,
            gs0, gs1, gs2, gs3, ss0, ss1, ss2, ss3):
    c = lax.axis_index("c")
    s = lax.axis_index("s")
    w = c * NSUB + s
    pltpu.sync_copy(src_hbm.at[w], src_v)
    pltpu.sync_copy(dst_hbm.at[w], dst_v)
    pltpu.sync_copy(zeros_hbm.at[pl.ds(s * ROWS_PT, ROWS_PT)],
                    acc_sh.at[pl.ds(s * ROWS_PT, ROWS_PT)])
    pltpu.sync_copy(table_hbm.at[pl.ds(s * (N // NSUB), N // NSUB)],
                    table_sh.at[pl.ds(s * (N // NSUB), N // NSUB)])
    plsc.subcore_barrier()

    R = (rows0_v, rows1_v, rows2_v, rows3_v)
    GS = (gs0, gs1, gs2, gs3)
    SS = (ss0, ss1, ss2, ss3)

    def _gi(i, b):
        pltpu.async_copy(table_sh.at[src_v.at[i]], R[b], GS[b])

    def _gw(i, b):
        pltpu.make_async_copy(table_sh.at[src_v.at[i]], R[b], GS[b]).wait()

    def _si(i, b):
        pltpu.async_copy(R[b], acc_sh.at[dst_v.at[i]], SS[b], add=True)

    def _sw(i, b):
        pltpu.make_async_copy(R[b], acc_sh.at[dst_v.at[i]], SS[b]).wait()

    _gi(0, 0)
    _gi(1, 1)
    _gw(0, 0)
    _si(0, 0)
    _gi(2, 2)
    _gw(1, 1)
    _si(1, 1)
    _gi(3, 3)

    def body(k, carry):
        i0 = 2 + 4 * k
        for u in range(4):
            i = i0 + u
            b = (2 + u) % 4
            _gw(i, b)
            _si(i, b)
            b2 = (b + 2) % 4
            _sw(i - 2, b2)
            _gi(i + 2, b2)
        return carry

    lax.fori_loop(0, (NCHUNK - 5) // 4, body, 0)
    _gw(NCHUNK - 3, 2)
    _si(NCHUNK - 3, 2)
    _sw(NCHUNK - 5, 0)
    _gi(NCHUNK - 1, 0)
    _gw(NCHUNK - 2, 3)
    _si(NCHUNK - 2, 3)
    _gw(NCHUNK - 1, 0)
    _si(NCHUNK - 1, 0)
    _sw(NCHUNK - 4, 1)
    _sw(NCHUNK - 3, 2)
    _sw(NCHUNK - 2, 3)
    _sw(NCHUNK - 1, 0)
    plsc.subcore_barrier()
    pltpu.sync_copy(acc_sh.at[pl.ds(s * ROWS_PT, ROWS_PT)],
                    out_hbm.at[c, pl.ds(s * ROWS_PT, ROWS_PT)])


def _din(dc0_ref, dc1_ref):
    deg = dc0_ref[...][:N] + dc1_ref[...][:N] + 1.0
    return lax.rsqrt(deg[:, :1])


def _tc_first_body(x_ref, w_ref, dc0_ref, dc1_ref, hwp_ref):
    din = _din(dc0_ref, dc1_ref)
    hw = jnp.dot(x_ref[...], w_ref[...], preferred_element_type=jnp.float32)
    hwp_ref[...] = hw * din


_tc_first = pl.pallas_call(
    _tc_first_body,
    out_shape=jax.ShapeDtypeStruct((N, H), jnp.float32),
)


def _tc_mid_body(acc0_ref, acc1_ref, hwp_ref, dc0_ref, dc1_ref, b_ref, w_ref,
                 out_ref):
    din = _din(dc0_ref, dc1_ref)
    h = jnp.tanh(din * (acc0_ref[...][:N] + acc1_ref[...][:N] + hwp_ref[...])
                 + b_ref[...])
    out_ref[...] = jnp.dot(h, w_ref[...],
                           preferred_element_type=jnp.float32) * din


_tc_mid = pl.pallas_call(
    _tc_mid_body,
    out_shape=jax.ShapeDtypeStruct((N, H), jnp.float32),
)


def _tc_pool_body(acc0_ref, acc1_ref, hwp_ref, dc0_ref, dc1_ref, b_ref,
                  bi_ref, wout_ref, bout_ref, out_ref, hidden_ref):
    din = _din(dc0_ref, dc1_ref)
    h = jnp.tanh(din * (acc0_ref[...][:N] + acc1_ref[...][:N] + hwp_ref[...])
                 + b_ref[...])
    bi = bi_ref[...]
    onehot = (bi == lax.broadcasted_iota(jnp.int32, (1, G), 1)
              ).astype(jnp.float32)
    sums = lax.dot_general(onehot, h, (((0,), (0,)), ((), ())),
                           preferred_element_type=jnp.float32)
    counts = lax.dot_general(onehot, jnp.ones((N, 1), jnp.float32),
                             (((0,), (0,)), ((), ())),
                             preferred_element_type=jnp.float32)
    neg = jnp.float32(-jnp.inf)

    m = h
    d = 1
    while d < N:
        ids_sh = jnp.concatenate(
            [jnp.full((d, 1), -1, jnp.int32), bi[:N - d]], axis=0)
        m_sh = jnp.concatenate(
            [jnp.full((d, H), neg, jnp.float32), m[:N - d]], axis=0)
        m = jnp.maximum(m, jnp.where(ids_sh == bi, m_sh, neg))
        d *= 2
    nxt = jnp.concatenate([bi[1:], jnp.full((1, 1), -1, jnp.int32)], axis=0)
    wlast = onehot * (bi != nxt).astype(jnp.float32)
    gmp = lax.dot_general(wlast, m, (((0,), (0,)), ((), ())),
                          preferred_element_type=jnp.float32)
    gmp = jnp.where(counts > 0.0, gmp, 0.0)
    gap = sums / jnp.maximum(counts, 1.0)
    hidden = jnp.concatenate([gmp, gap], axis=1)
    out_ref[...] = jnp.dot(hidden, wout_ref[...],
                           preferred_element_type=jnp.float32) + bout_ref[...]
    hidden_ref[...] = hidden


_tc_pool = pl.pallas_call(
    _tc_pool_body,
    out_shape=(jax.ShapeDtypeStruct((G, 1), jnp.float32),
               jax.ShapeDtypeStruct((G, 2 * H), jnp.float32)),
    compiler_params=pltpu.CompilerParams(vmem_limit_bytes=100 * 1024 * 1024),
)


def kernel(x, edge_index, batch_index, W0, b0, W1, b1, W2, b2, W3, b3,
           Wout, bout):
    src = edge_index[0].reshape(NW, NCHUNK, CH)
    dst = edge_index[1].reshape(NW, NCHUNK, CH)
    ones16 = jnp.ones((CH, 16), jnp.float32)
    zeros16 = jnp.zeros((NP, 16), jnp.float32)
    zerosH = jnp.zeros((NP, H), jnp.float32)

    degcnt = _sc_degree(dst, ones16, zeros16)
    dc0, dc1 = degcnt[0], degcnt[1]

    hwp = _tc_first(x, W0, dc0, dc1)
    for (b, W) in ((b0, W1), (b1, W2), (b2, W3)):
        acc = _sc_agg(hwp, src, dst, zerosH)
        hwp = _tc_mid(acc[0], acc[1], hwp, dc0, dc1, b.reshape(1, H), W)
    acc = _sc_agg(hwp, src, dst, zerosH)
    out, hidden = _tc_pool(acc[0], acc[1], hwp, dc0, dc1, b3.reshape(1, H),
                           batch_index.reshape(N, 1), Wout,
                           bout.reshape(1, 1))
    return (out, hidden)


# speedup vs baseline: 29.5939x; 1.0124x over previous
import functools

import jax
import jax.numpy as jnp
from jax import lax
from jax.experimental import pallas as pl
from jax.experimental.pallas import tpu as pltpu
from jax.experimental.pallas import tpu_sc as plsc

N = 10000
E = 320000
F = 128
H = 64
G = 128

NCORE = 2
NSUB = 16
NW = NCORE * NSUB
EPT = E // NW
CH = 80
NCHUNK = EPT // CH
NP = 10112
ROWS_PT = NP // NSUB

_MESH = dict(core_axis_name="c", subcore_axis_name="s")


@functools.partial(
    pl.kernel,
    mesh=plsc.VectorSubcoreMesh(**_MESH),
    compiler_params=pltpu.CompilerParams(use_tc_tiling_on_sc=False),
    out_type=jax.ShapeDtypeStruct((NCORE, NP, 16), jnp.float32),
    scratch_types=[
        pltpu.VMEM((NCHUNK, CH), jnp.int32),
        pltpu.VMEM((CH, 16), jnp.float32),
        pltpu.VMEM_SHARED((NP, 16), jnp.float32),
        pltpu.SemaphoreType.DMA,
        pltpu.SemaphoreType.DMA,
        pltpu.SemaphoreType.DMA,
        pltpu.SemaphoreType.DMA,
    ],
)
def _sc_degree(dst_hbm, ones_hbm, zeros_hbm, out_hbm, dst_v, ones_v, acc_sh,
               ds0, ds1, ds2, ds3):
    c = lax.axis_index("c")
    s = lax.axis_index("s")
    w = c * NSUB + s
    pltpu.sync_copy(dst_hbm.at[w], dst_v)
    pltpu.sync_copy(ones_hbm, ones_v)
    pltpu.sync_copy(zeros_hbm.at[pl.ds(s * ROWS_PT, ROWS_PT)],
                    acc_sh.at[pl.ds(s * ROWS_PT, ROWS_PT)])
    plsc.subcore_barrier()

    DS = (ds0, ds1, ds2, ds3)

    def body(k, carry):
        j0 = 4 * k
        for u in range(4):
            pltpu.async_copy(ones_v, acc_sh.at[dst_v.at[j0 + u]], DS[u],
                             add=True)
        for u in range(4):
            pltpu.make_async_copy(ones_v, acc_sh.at[dst_v.at[j0 + u]],
                                  DS[u]).wait()
        return carry

    lax.fori_loop(0, NCHUNK // 4, body, 0)
    pltpu.sync_copy(ones_v, acc_sh.at[dst_v.at[NCHUNK - 1]], add=True)
    plsc.subcore_barrier()
    pltpu.sync_copy(acc_sh.at[pl.ds(s * ROWS_PT, ROWS_PT)],
                    out_hbm.at[c, pl.ds(s * ROWS_PT, ROWS_PT)])


@functools.partial(
    pl.kernel,
    mesh=plsc.VectorSubcoreMesh(**_MESH),
    compiler_params=pltpu.CompilerParams(use_tc_tiling_on_sc=False),
    out_type=jax.ShapeDtypeStruct((NCORE, NP, H), jnp.float32),
    scratch_types=[
        pltpu.VMEM((NCHUNK, CH), jnp.int32),
        pltpu.VMEM((NCHUNK, CH), jnp.int32),
        pltpu.VMEM((CH, H), jnp.float32),
        pltpu.VMEM((CH, H), jnp.float32),
        pltpu.VMEM((CH, H), jnp.float32),
        pltpu.VMEM((CH, H), jnp.float32),
        pltpu.VMEM_SHARED((NP, H), jnp.float32),
        pltpu.VMEM_SHARED((N, H), jnp.float32),
        pltpu.SemaphoreType.DMA,
        pltpu.SemaphoreType.DMA,
        pltpu.SemaphoreType.DMA,
        pltpu.SemaphoreType.DMA,
        pltpu.SemaphoreType.DMA,
        pltpu.SemaphoreType.DMA,
        pltpu.SemaphoreType.DMA,
        pltpu.SemaphoreType.DMA,
    ],
)
def _sc_agg(table_hbm, src_hbm, dst_hbm, zeros_hbm, out_hbm,
            src_v, dst_v, rows0_v, rows1_v, rows2_v, rows3_v,
            acc_sh, table_sh,
            gs0, gs1, gs2, gs3, ss0, ss1, ss2, ss3):
    c = lax.axis_index("c")
    s = lax.axis_index("s")
    w = c * NSUB + s
    pltpu.sync_copy(src_hbm.at[w], src_v)
    pltpu.sync_copy(dst_hbm.at[w], dst_v)
    pltpu.sync_copy(zeros_hbm.at[pl.ds(s * ROWS_PT, ROWS_PT)],
                    acc_sh.at[pl.ds(s * ROWS_PT, ROWS_PT)])
    pltpu.sync_copy(table_hbm.at[pl.ds(s * (N // NSUB), N // NSUB)],
                    table_sh.at[pl.ds(s * (N // NSUB), N // NSUB)])
    plsc.subcore_barrier()

    R = (rows0_v, rows1_v, rows2_v, rows3_v)
    GS = (gs0, gs1, gs2, gs3)
    SS = (ss0, ss1, ss2, ss3)

    def _gi(i, b):
        pltpu.async_copy(table_sh.at[src_v.at[i]], R[b], GS[b])

    def _gw(i, b):
        pltpu.make_async_copy(table_sh.at[src_v.at[i]], R[b], GS[b]).wait()

    def _si(i, b):
        pltpu.async_copy(R[b], acc_sh.at[dst_v.at[i]], SS[b], add=True)

    def _sw(i, b):
        pltpu.make_async_copy(R[b], acc_sh.at[dst_v.at[i]], SS[b]).wait()

    _gi(0, 0)
    _gi(1, 1)
    _gw(0, 0)
    _si(0, 0)
    _gi(2, 2)
    _gw(1, 1)
    _si(1, 1)
    _gi(3, 3)

    def body(k, carry):
        i0 = 2 + 4 * k
        for u in range(4):
            i = i0 + u
            b = (2 + u) % 4
            _gw(i, b)
            _si(i, b)
            b2 = (b + 2) % 4
            _sw(i - 2, b2)
            _gi(i + 2, b2)
        return carry

    lax.fori_loop(0, (NCHUNK - 5) // 4, body, 0)
    _gw(NCHUNK - 3, 2)
    _si(NCHUNK - 3, 2)
    _sw(NCHUNK - 5, 0)
    _gi(NCHUNK - 1, 0)
    _gw(NCHUNK - 2, 3)
    _si(NCHUNK - 2, 3)
    _gw(NCHUNK - 1, 0)
    _si(NCHUNK - 1, 0)
    _sw(NCHUNK - 4, 1)
    _sw(NCHUNK - 3, 2)
    _sw(NCHUNK - 2, 3)
    _sw(NCHUNK - 1, 0)
    plsc.subcore_barrier()
    pltpu.sync_copy(acc_sh.at[pl.ds(s * ROWS_PT, ROWS_PT)],
                    out_hbm.at[c, pl.ds(s * ROWS_PT, ROWS_PT)])


def _din(dc0_ref, dc1_ref):
    deg = dc0_ref[...][:N] + dc1_ref[...][:N] + 1.0
    return lax.rsqrt(deg[:, :1])


def _tc_first_body(x_ref, w_ref, dc0_ref, dc1_ref, hwp_ref):
    din = _din(dc0_ref, dc1_ref)
    hw = jnp.dot(x_ref[...], w_ref[...], preferred_element_type=jnp.float32)
    hwp_ref[...] = hw * din


_tc_first = pl.pallas_call(
    _tc_first_body,
    out_shape=jax.ShapeDtypeStruct((N, H), jnp.float32),
)


def _tc_mid_body(acc0_ref, acc1_ref, hwp_ref, dc0_ref, dc1_ref, b_ref, w_ref,
                 out_ref):
    din = _din(dc0_ref, dc1_ref)
    h = jnp.tanh(din * (acc0_ref[...][:N] + acc1_ref[...][:N] + hwp_ref[...])
                 + b_ref[...])
    out_ref[...] = jnp.dot(h, w_ref[...],
                           preferred_element_type=jnp.float32) * din


_tc_mid = pl.pallas_call(
    _tc_mid_body,
    out_shape=jax.ShapeDtypeStruct((N, H), jnp.float32),
)


def _tc_pool_body(acc0_ref, acc1_ref, hwp_ref, dc0_ref, dc1_ref, b_ref,
                  bi_ref, wout_ref, bout_ref, out_ref, hidden_ref):
    din = _din(dc0_ref, dc1_ref)
    h = jnp.tanh(din * (acc0_ref[...][:N] + acc1_ref[...][:N] + hwp_ref[...])
                 + b_ref[...])
    bi = bi_ref[...]
    onehot = (bi == lax.broadcasted_iota(jnp.int32, (1, G), 1)
              ).astype(jnp.float32)
    sums = lax.dot_general(onehot, h, (((0,), (0,)), ((), ())),
                           preferred_element_type=jnp.float32)
    counts = lax.dot_general(onehot, jnp.ones((N, 1), jnp.float32),
                             (((0,), (0,)), ((), ())),
                             preferred_element_type=jnp.float32)
    neg = jnp.float32(-jnp.inf)

    m = h
    d = 1
    while d < N:
        ids_sh = jnp.concatenate(
            [jnp.full((d, 1), -1, jnp.int32), bi[:N - d]], axis=0)
        m_sh = jnp.concatenate(
            [jnp.full((d, H), neg, jnp.float32), m[:N - d]], axis=0)
        m = jnp.maximum(m, jnp.where(ids_sh == bi, m_sh, neg))
        d *= 2
    nxt = jnp.concatenate([bi[1:], jnp.full((1, 1), -1, jnp.int32)], axis=0)
    wlast = onehot * (bi != nxt).astype(jnp.float32)
    gmp = lax.dot_general(wlast, m, (((0,), (0,)), ((), ())),
                          preferred_element_type=jnp.float32)
    gmp = jnp.where(counts > 0.0, gmp, 0.0)
    gap = sums / jnp.maximum(counts, 1.0)
    hidden = jnp.concatenate([gmp, gap], axis=1)
    out_ref[...] = jnp.dot(hidden, wout_ref[...],
                           preferred_element_type=jnp.float32) + bout_ref[...]
    hidden_ref[...] = hidden


_tc_pool = pl.pallas_call(
    _tc_pool_body,
    out_shape=(jax.ShapeDtypeStruct((G, 1), jnp.float32),
               jax.ShapeDtypeStruct((G, 2 * H), jnp.float32)),
    compiler_params=pltpu.CompilerParams(vmem_limit_bytes=100 * 1024 * 1024),
)


def kernel(x, edge_index, batch_index, W0, b0, W1, b1, W2, b2, W3, b3,
           Wout, bout):
    src = edge_index[0].reshape(NW, NCHUNK, CH)
    dst = edge_index[1].reshape(NW, NCHUNK, CH)
    ones16 = jnp.ones((CH, 16), jnp.float32)
    zeros16 = jnp.zeros((NP, 16), jnp.float32)
    zerosH = jnp.zeros((NP, H), jnp.float32)

    degcnt = _sc_degree(dst, ones16, zeros16)
    dc0, dc1 = degcnt[0], degcnt[1]

    hwp = _tc_first(x, W0, dc0, dc1)
    for (b, W) in ((b0, W1), (b1, W2), (b2, W3)):
        acc = _sc_agg(hwp, src, dst, zerosH)
        hwp = _tc_mid(acc[0], acc[1], hwp, dc0, dc1, b.reshape(1, H), W)
    acc = _sc_agg(hwp, src, dst, zerosH)
    out, hidden = _tc_pool(acc[0], acc[1], hwp, dc0, dc1, b3.reshape(1, H),
                           batch_index.reshape(N, 1), Wout,
                           bout.reshape(1, 1))
    return (out, hidden)
